# baseline (speedup 1.0000x reference)
import functools

import jax
import jax.numpy as jnp
from jax import lax
from jax.experimental import pallas as pl
from jax.experimental.pallas import tpu as pltpu
from jax.experimental.pallas import tpu_sc as plsc

N_NODES = 10000
N_EDGES = 320000
D_LAT = 128

NC = 2
NS = 16
NW = NC * NS
CSZ = 128
NCHUNK = 80
EPW = NCHUNK * CSZ
E_PAD = NW * EPW
N_ACC = 10016
TRASH = N_NODES
RPT = 624
REM = N_ACC - NS * RPT
REM_OFF = NS * RPT

F32 = jnp.float32
BF16 = jnp.bfloat16


def _mesh():
    return plsc.VectorSubcoreMesh(
        core_axis_name="c", subcore_axis_name="s", num_cores=NC, num_subcores=NS
    )


@functools.cache
def _build_sc_gather(nchunk):
    epw = nchunk * CSZ
    ne = NW * epw

    @functools.partial(
        pl.kernel,
        out_type=[
            jax.ShapeDtypeStruct((ne, D_LAT // 2), jnp.int32),
            jax.ShapeDtypeStruct((ne, D_LAT // 2), jnp.int32),
        ],
        mesh=_mesh(),
        scratch_types=[
            pltpu.VMEM((nchunk, CSZ), jnp.int32),
            pltpu.VMEM((nchunk, CSZ), jnp.int32),
            pltpu.VMEM((CSZ, D_LAT // 2), jnp.int32),
            pltpu.VMEM((CSZ, D_LAT // 2), jnp.int32),
            pltpu.VMEM((CSZ, D_LAT // 2), jnp.int32),
            pltpu.VMEM((CSZ, D_LAT // 2), jnp.int32),
            pltpu.SemaphoreType.DMA,
            pltpu.SemaphoreType.DMA,
            pltpu.SemaphoreType.DMA,
            pltpu.SemaphoreType.DMA,
        ],
        compiler_params=pltpu.CompilerParams(use_tc_tiling_on_sc=False),
    )
    def gather(xd_hbm, xs_hbm, dst_hbm, src_hbm, g1_hbm, g2_hbm,
               idxd_v, idxs_v, r1a, r1b, r2a, r2b, gsa, gsb, wsa, wsb):
        wid = lax.axis_index("s") * NC + lax.axis_index("c")
        base = wid * epw
        R1 = (r1a, r1b)
        R2 = (r2a, r2b)
        GS = (gsa, gsb)
        WS = (wsa, wsb)
        pltpu.sync_copy(dst_hbm.at[wid], idxd_v)
        pltpu.sync_copy(src_hbm.at[wid], idxs_v)

        def fire_gather(k, b):
            pltpu.async_copy(xd_hbm.at[idxd_v.at[k]], R1[b], GS[b])
            pltpu.async_copy(xs_hbm.at[idxs_v.at[k]], R2[b], GS[b])

        def drain_gather(b):
            pltpu.make_async_copy(xd_hbm.at[idxd_v.at[0]], R1[b], GS[b]).wait()
            pltpu.make_async_copy(xs_hbm.at[idxs_v.at[0]], R2[b], GS[b]).wait()

        def fire_write(k, b):
            off = base + k * CSZ
            pltpu.async_copy(R1[b], g1_hbm.at[pl.ds(off, CSZ)], WS[b])
            pltpu.async_copy(R2[b], g2_hbm.at[pl.ds(off, CSZ)], WS[b])

        def drain_write(b):
            pltpu.make_async_copy(R1[b], g1_hbm.at[pl.ds(base, CSZ)], WS[b]).wait()
            pltpu.make_async_copy(R2[b], g2_hbm.at[pl.ds(base, CSZ)], WS[b]).wait()

        fire_gather(0, 0)

        @pl.loop(0, nchunk, step=2)
        def _pair(j):
            for b in range(2):
                k = j + b
                o = 1 - b
                drain_gather(b)

                @pl.when(k > 0)
                def _():
                    drain_write(o)

                @pl.when(k + 1 < nchunk)
                def _():
                    fire_gather(k + 1, o)

                fire_write(k, b)

        drain_write((nchunk - 1) % 2)

    return gather


def _sc_gather(xd, xs, dstg3, srcg3):
    return _build_sc_gather(dstg3.shape[1])(xd, xs, dstg3, srcg3)


@functools.cache
def _build_sc_scatter(nchunk):
    epw = nchunk * CSZ

    @functools.partial(
        pl.kernel,
        out_type=jax.ShapeDtypeStruct((NC, N_ACC, D_LAT), F32),
        mesh=_mesh(),
        scratch_types=[
            pltpu.VMEM((nchunk, CSZ), jnp.int32),
            pltpu.VMEM((CSZ, D_LAT), F32),
            pltpu.VMEM((CSZ, D_LAT), F32),
            pltpu.VMEM_SHARED((N_ACC, D_LAT), F32),
            pltpu.SemaphoreType.DMA,
            pltpu.SemaphoreType.DMA,
            pltpu.SemaphoreType.DMA,
            pltpu.SemaphoreType.DMA,
        ],
    )
    def scatter(e_hbm, dst_hbm, z_hbm, out_hbm,
                idx_v, ra, rb, acc_sh, rsa, rsb, asa, asb):
        c = lax.axis_index("c")
        s = lax.axis_index("s")
        wid = s * NC + c
        base = wid * epw
        R = (ra, rb)
        RS = (rsa, rsb)
        AS = (asa, asb)

        def fire_read(k, b):
            pltpu.async_copy(e_hbm.at[pl.ds(base + k * CSZ, CSZ)], R[b], RS[b])

        def drain_read(b):
            pltpu.make_async_copy(e_hbm.at[pl.ds(base, CSZ)], R[b], RS[b]).wait()

        def fire_add(k, b):
            pltpu.async_copy(R[b], acc_sh.at[idx_v.at[k]], AS[b], add=True)

        def drain_add(b):
            pltpu.make_async_copy(R[b], acc_sh.at[idx_v.at[0]], AS[b]).wait()

        fire_read(0, 0)
        pltpu.sync_copy(dst_hbm.at[wid], idx_v)
        pltpu.sync_copy(z_hbm.at[pl.ds(s * RPT, RPT)],
                        acc_sh.at[pl.ds(s * RPT, RPT)])

        @pl.when(s == NS - 1)
        def _zero_rem():
            pltpu.sync_copy(z_hbm.at[pl.ds(REM_OFF, REM)],
                            acc_sh.at[pl.ds(REM_OFF, REM)])

        plsc.subcore_barrier()

        @pl.loop(0, nchunk, step=2)
        def _pair(j):
            for b in range(2):
                k = j + b
                o = 1 - b
                drain_read(b)

                @pl.when(k > 0)
                def _():
                    drain_add(o)

                @pl.when(k + 1 < nchunk)
                def _():
                    fire_read(k + 1, o)

                fire_add(k, b)

        drain_add((nchunk - 1) % 2)
        plsc.subcore_barrier()
        pltpu.sync_copy(acc_sh.at[pl.ds(s * RPT, RPT)],
                        out_hbm.at[c, pl.ds(s * RPT, RPT)])

        @pl.when(s == NS - 1)
        def _write_rem():
            pltpu.sync_copy(acc_sh.at[pl.ds(REM_OFF, REM)],
                            out_hbm.at[c, pl.ds(REM_OFF, REM)])

    return scatter


def _sc_scatter(e, dsts3, zeros_init):
    return _build_sc_scatter(dsts3.shape[1])(e, dsts3, zeros_init)


T_NODE = 2000
T_EDGE = 2048


def _dot(a, b):
    return jnp.dot(a, b, preferred_element_type=F32)


def _node_enc_body(nf, w1, b1, w2, b2, w3, b3, wd, ws, x_o, xd_o, xs_o):
    h = jnp.maximum(_dot(nf[...], w1[...]) + b1[...], 0.0)
    h = jnp.maximum(_dot(h, w2[...]) + b2[...], 0.0)
    x = _dot(h, w3[...]) + b3[...]
    x_o[...] = x
    xd_o[...] = _dot(x, wd[...]).astype(BF16)
    xs_o[...] = _dot(x, ws[...]).astype(BF16)


def _edge_enc_body(ef, w1, b1, w2, b2, w3, b3, e_o):
    h = jnp.maximum(_dot(ef[...], w1[...]) + b1[...], 0.0)
    h = jnp.maximum(_dot(h, w2[...]) + b2[...], 0.0)
    e_o[...] = _dot(h, w3[...]) + b3[...]


def _unpack_perm(w):
    lo = lax.bitcast_convert_type(w << 16, F32)
    hi = lax.bitcast_convert_type(w & jnp.int32(-65536), F32)
    return jnp.concatenate([lo, hi], axis=1)


def _edge_mlp_body(g1, g2, e, we, b1, w2, b2, w3, b3, e_o):
    ga = _unpack_perm(g1[...])
    gb = _unpack_perm(g2[...])
    h = ga + gb + _dot(e[...], we[...]) + b1[...]
    h = jnp.maximum(h, 0.0)
    h = jnp.maximum(_dot(h, w2[...]) + b2[...], 0.0)
    e_o[...] = _dot(h, w3[...]) + b3[...]


def _node_upd_proj_body(x, ag, ag2, wnx, wna, b1, w2, b2, w3, b3, wd, ws,
                        x_o, xd_o, xs_o):
    a = ag[0] + ag[1] + ag2[0] + ag2[1]
    h = jnp.maximum(_dot(x[...], wnx[...]) + _dot(a, wna[...]) + b1[...], 0.0)
    h = jnp.maximum(_dot(h, w2[...]) + b2[...], 0.0)
    xn = _dot(h, w3[...]) + b3[...]
    x_o[...] = xn
    xd_o[...] = _dot(xn, wd[...]).astype(BF16)
    xs_o[...] = _dot(xn, ws[...]).astype(BF16)


def _node_upd_dec_body(x, ag, ag2, wnx, wna, b1, w2, b2, w3, b3,
                       dw1, db1, dw2, db2, dw3, db3, out_o):
    a = ag[0] + ag[1] + ag2[0] + ag2[1]
    h = jnp.maximum(_dot(x[...], wnx[...]) + _dot(a, wna[...]) + b1[...], 0.0)
    h = jnp.maximum(_dot(h, w2[...]) + b2[...], 0.0)
    xn = _dot(h, w3[...]) + b3[...]
    d = jnp.maximum(_dot(xn, dw1[...]) + db1[...], 0.0)
    d = jnp.maximum(_dot(d, dw2[...]) + db2[...], 0.0)
    out_o[...] = _dot(d, dw3[...]) + db3[...]


def _row_spec(t, width):
    return pl.BlockSpec((t, width), lambda i: (i, 0))


def _w_spec(shape):
    nz = (0,) * len(shape)
    return pl.BlockSpec(shape, lambda i, _nz=nz: _nz)


def _tc_call(body, grid, in_specs, out_specs, out_shape):
    return pl.pallas_call(
        body,
        grid=(grid,),
        in_specs=in_specs,
        out_specs=out_specs,
        out_shape=out_shape,
    )


def _wb(layer):
    return layer["W"], layer["b"].reshape(1, -1)


def _pack(a):
    return lax.bitcast_convert_type(
        a.reshape(a.shape[0], D_LAT // 2, 2), jnp.int32)


_PERM = tuple(range(0, D_LAT, 2)) + tuple(range(1, D_LAT, 2))
_PERM = jnp.array(_PERM, jnp.int32)


def kernel(node_features, edge_features, edge_index, params):
    src = edge_index[0]
    dst = edge_index[1]
    npad = E_PAD - N_EDGES
    pad_ids = (jnp.arange(npad, dtype=jnp.int32) * 8) % N_NODES
    eh = E_PAD // 2
    nchunk_h = eh // NW // CSZ
    dstg = jnp.concatenate([dst, pad_ids])
    srcg = jnp.concatenate([src, pad_ids])
    dsts = jnp.concatenate([dst, jnp.full((npad,), TRASH, jnp.int32)])
    dstg3 = [dstg[h * eh:(h + 1) * eh].reshape(NW, nchunk_h, CSZ)
             for h in range(2)]
    srcg3 = [srcg[h * eh:(h + 1) * eh].reshape(NW, nchunk_h, CSZ)
             for h in range(2)]
    dsts3 = [dsts[h * eh:(h + 1) * eh].reshape(NW, nchunk_h, CSZ)
             for h in range(2)]
    ef_pad = jnp.pad(edge_features, ((0, npad), (0, 0)))
    ef_h = [ef_pad[h * eh:(h + 1) * eh] for h in range(2)]
    zeros_init = jnp.zeros((N_ACC, D_LAT), F32)

    en = params["enc_node"]
    ee = params["enc_edge"]
    proc = params["proc"]
    dec = params["dec"]

    def esplit(p):
        w1 = p[0]["W"]
        return w1[0:D_LAT], w1[D_LAT:2 * D_LAT], w1[2 * D_LAT:3 * D_LAT]

    def nsplit(p):
        w1 = p[0]["W"]
        return w1[0:D_LAT], w1[D_LAT:2 * D_LAT]

    wsp = _w_spec((D_LAT, D_LAT))
    bsp = _w_spec((1, D_LAT))

    wd0, ws0, _ = esplit(proc[0]["edge"])
    w1, b1 = _wb(en[0]); w2, b2 = _wb(en[1]); w3, b3 = _wb(en[2])
    ng = N_NODES // T_NODE
    x, xd, xs = _tc_call(
        _node_enc_body, ng,
        [_row_spec(T_NODE, D_LAT)] + [wsp, bsp] * 3 + [wsp, wsp],
        [_row_spec(T_NODE, D_LAT)] * 3,
        [jax.ShapeDtypeStruct((N_NODES, D_LAT), F32),
         jax.ShapeDtypeStruct((N_NODES, D_LAT), BF16),
         jax.ShapeDtypeStruct((N_NODES, D_LAT), BF16)],
    )(node_features, w1, b1, w2, b2, w3, b3, wd0, ws0)
    xd = _pack(xd)
    xs = _pack(xs)

    w1, b1 = _wb(ee[0]); w2, b2 = _wb(ee[1]); w3, b3 = _wb(ee[2])
    eg = eh // T_EDGE
    e_h = [
        _tc_call(
            _edge_enc_body, eg,
            [_row_spec(T_EDGE, 16), _w_spec((16, D_LAT)), bsp, wsp, bsp,
             wsp, bsp],
            _row_spec(T_EDGE, D_LAT),
            jax.ShapeDtypeStruct((eh, D_LAT), F32),
        )(ef_h[h], w1, b1, w2, b2, w3, b3)
        for h in range(2)
    ]

    n_steps = len(proc)
    out = None
    for i in range(n_steps):
        pe = proc[i]["edge"]
        pn = proc[i]["node"]
        _, _, we = esplit(pe)
        we = we[:, _PERM]
        b1e = pe[0]["b"][_PERM].reshape(1, -1)
        w2e, b2e = _wb(pe[1])
        w2e = w2e[_PERM, :]
        w3e, b3e = _wb(pe[2])

        g_h = [_sc_gather(xd, xs, dstg3[h], srcg3[h]) for h in range(2)]
        ag_h = []
        for h in range(2):
            g1, g2 = g_h[h]
            e_h[h] = _tc_call(
                _edge_mlp_body, eg,
                [_row_spec(T_EDGE, D_LAT // 2)] * 2
                + [_row_spec(T_EDGE, D_LAT), wsp, bsp, wsp, bsp, wsp, bsp],
                _row_spec(T_EDGE, D_LAT),
                jax.ShapeDtypeStruct((eh, D_LAT), F32),
            )(g1, g2, e_h[h], we, b1e, w2e, b2e, w3e, b3e)
            ag_h.append(_sc_scatter(e_h[h], dsts3[h], zeros_init))

        wnx, wna = nsplit(pn)
        b1n = pn[0]["b"].reshape(1, -1)
        w2n, b2n = _wb(pn[1]); w3n, b3n = _wb(pn[2])
        agspec = pl.BlockSpec((NC, T_NODE, D_LAT), lambda i: (0, i, 0))

        if i + 1 < n_steps:
            wd1, ws1, _ = esplit(proc[i + 1]["edge"])
            x, xd, xs = _tc_call(
                _node_upd_proj_body, ng,
                [_row_spec(T_NODE, D_LAT), agspec, agspec,
                 wsp, wsp, bsp, wsp, bsp, wsp, bsp, wsp, wsp],
                [_row_spec(T_NODE, D_LAT)] * 3,
                [jax.ShapeDtypeStruct((N_NODES, D_LAT), F32),
                 jax.ShapeDtypeStruct((N_NODES, D_LAT), BF16),
                 jax.ShapeDtypeStruct((N_NODES, D_LAT), BF16)],
            )(x, ag_h[0], ag_h[1], wnx, wna, b1n, w2n, b2n, w3n, b3n,
              wd1, ws1)
            xd = _pack(xd)
            xs = _pack(xs)
        else:
            dw1, db1 = _wb(dec[0]); dw2, db2 = _wb(dec[1]); dw3, db3 = _wb(dec[2])
            out = _tc_call(
                _node_upd_dec_body, ng,
                [_row_spec(T_NODE, D_LAT), agspec, agspec,
                 wsp, wsp, bsp, wsp, bsp, wsp, bsp,
                 wsp, bsp, wsp, bsp, _w_spec((D_LAT, 3)), _w_spec((1, 3))],
                _row_spec(T_NODE, 3),
                jax.ShapeDtypeStruct((N_NODES, 3), F32),
            )(x, ag_h[0], ag_h[1], wnx, wna, b1n, w2n, b2n, w3n, b3n,
              dw1, db1, dw2, db2, dw3, db3)
    return out


# speedup vs baseline: 1.5111x; 1.5111x over previous
import functools

import jax
import jax.numpy as jnp
from jax import lax
from jax.experimental import pallas as pl
from jax.experimental.pallas import tpu as pltpu
from jax.experimental.pallas import tpu_sc as plsc

N_NODES = 10000
N_EDGES = 320000
D_LAT = 128

NC = 2
NS = 16
NW = NC * NS
CSZ = 128
NCHUNK = 80
EPW = NCHUNK * CSZ
E_PAD = NW * EPW
N_ACC = 10016
TRASH = N_NODES
RPT = 624
REM = N_ACC - NS * RPT
REM_OFF = NS * RPT

F32 = jnp.float32
BF16 = jnp.bfloat16


def _mesh():
    return plsc.VectorSubcoreMesh(
        core_axis_name="c", subcore_axis_name="s", num_cores=NC, num_subcores=NS
    )


@functools.cache
def _build_sc_gather(nchunk):
    epw = nchunk * CSZ
    ne = NW * epw

    @functools.partial(
        pl.kernel,
        out_type=jax.ShapeDtypeStruct((ne, D_LAT), F32),
        mesh=_mesh(),
        scratch_types=[
            pltpu.VMEM((nchunk, CSZ), jnp.int32),
            pltpu.VMEM((nchunk, CSZ), jnp.int32),
            pltpu.VMEM((CSZ, D_LAT), F32),
            pltpu.VMEM((CSZ, D_LAT), F32),
            pltpu.VMEM((CSZ, D_LAT), F32),
            pltpu.VMEM((CSZ, D_LAT), F32),
            pltpu.SemaphoreType.DMA,
            pltpu.SemaphoreType.DMA,
            pltpu.SemaphoreType.DMA,
            pltpu.SemaphoreType.DMA,
        ],
    )
    def gather(xd_hbm, xs_hbm, dst_hbm, src_hbm, g_hbm,
               idxd_v, idxs_v, r1a, r1b, r2a, r2b, gsa, gsb, wsa, wsb):
        wid = lax.axis_index("s") * NC + lax.axis_index("c")
        base = wid * epw
        R1 = (r1a, r1b)
        R2 = (r2a, r2b)
        GS = (gsa, gsb)
        WS = (wsa, wsb)
        pltpu.sync_copy(dst_hbm.at[wid], idxd_v)
        pltpu.sync_copy(src_hbm.at[wid], idxs_v)

        def fire_gather(k, b):
            pltpu.async_copy(xd_hbm.at[idxd_v.at[k]], R1[b], GS[b])
            pltpu.async_copy(xs_hbm.at[idxs_v.at[k]], R2[b], GS[b])

        def drain_gather(b):
            pltpu.make_async_copy(xd_hbm.at[idxd_v.at[0]], R1[b], GS[b]).wait()
            pltpu.make_async_copy(xs_hbm.at[idxs_v.at[0]], R2[b], GS[b]).wait()

        def fire_write(k, b):
            off = base + k * CSZ
            pltpu.async_copy(R1[b], g_hbm.at[pl.ds(off, CSZ)], WS[b])

        def drain_write(b):
            pltpu.make_async_copy(R1[b], g_hbm.at[pl.ds(base, CSZ)], WS[b]).wait()

        def add_rows(b):
            r1 = R1[b]
            r2 = R2[b]

            def row(r, carry):
                for cidx in range(D_LAT // 16):
                    sl = pl.ds(cidx * 16, 16)
                    r1[r, sl] = r1[r, sl] + r2[r, sl]
                return carry

            lax.fori_loop(0, CSZ, row, 0)

        fire_gather(0, 0)

        @pl.loop(0, nchunk, step=2)
        def _pair(j):
            for b in range(2):
                k = j + b
                o = 1 - b
                drain_gather(b)

                @pl.when(k > 0)
                def _():
                    drain_write(o)

                @pl.when(k + 1 < nchunk)
                def _():
                    fire_gather(k + 1, o)

                add_rows(b)
                fire_write(k, b)

        drain_write((nchunk - 1) % 2)

    return gather


def _sc_gather(xd, xs, dstg3, srcg3):
    return _build_sc_gather(dstg3.shape[1])(xd, xs, dstg3, srcg3)


@functools.cache
def _build_sc_scatter(nchunk):
    epw = nchunk * CSZ

    @functools.partial(
        pl.kernel,
        out_type=jax.ShapeDtypeStruct((NC, N_ACC, D_LAT), F32),
        mesh=_mesh(),
        scratch_types=[
            pltpu.VMEM((nchunk, CSZ), jnp.int32),
            pltpu.VMEM((CSZ, D_LAT), F32),
            pltpu.VMEM((CSZ, D_LAT), F32),
            pltpu.VMEM_SHARED((N_ACC, D_LAT), F32),
            pltpu.SemaphoreType.DMA,
            pltpu.SemaphoreType.DMA,
            pltpu.SemaphoreType.DMA,
            pltpu.SemaphoreType.DMA,
        ],
    )
    def scatter(e_hbm, dst_hbm, z_hbm, out_hbm,
                idx_v, ra, rb, acc_sh, rsa, rsb, asa, asb):
        c = lax.axis_index("c")
        s = lax.axis_index("s")
        wid = s * NC + c
        base = wid * epw
        R = (ra, rb)
        RS = (rsa, rsb)
        AS = (asa, asb)

        def fire_read(k, b):
            pltpu.async_copy(e_hbm.at[pl.ds(base + k * CSZ, CSZ)], R[b], RS[b])

        def drain_read(b):
            pltpu.make_async_copy(e_hbm.at[pl.ds(base, CSZ)], R[b], RS[b]).wait()

        def fire_add(k, b):
            pltpu.async_copy(R[b], acc_sh.at[idx_v.at[k]], AS[b], add=True)

        def drain_add(b):
            pltpu.make_async_copy(R[b], acc_sh.at[idx_v.at[0]], AS[b]).wait()

        fire_read(0, 0)
        pltpu.sync_copy(dst_hbm.at[wid], idx_v)
        pltpu.sync_copy(z_hbm.at[pl.ds(s * RPT, RPT)],
                        acc_sh.at[pl.ds(s * RPT, RPT)])

        @pl.when(s == NS - 1)
        def _zero_rem():
            pltpu.sync_copy(z_hbm.at[pl.ds(REM_OFF, REM)],
                            acc_sh.at[pl.ds(REM_OFF, REM)])

        plsc.subcore_barrier()

        @pl.loop(0, nchunk, step=2)
        def _pair(j):
            for b in range(2):
                k = j + b
                o = 1 - b
                drain_read(b)

                @pl.when(k > 0)
                def _():
                    drain_add(o)

                @pl.when(k + 1 < nchunk)
                def _():
                    fire_read(k + 1, o)

                fire_add(k, b)

        drain_add((nchunk - 1) % 2)
        plsc.subcore_barrier()
        pltpu.sync_copy(acc_sh.at[pl.ds(s * RPT, RPT)],
                        out_hbm.at[c, pl.ds(s * RPT, RPT)])

        @pl.when(s == NS - 1)
        def _write_rem():
            pltpu.sync_copy(acc_sh.at[pl.ds(REM_OFF, REM)],
                            out_hbm.at[c, pl.ds(REM_OFF, REM)])

    return scatter


def _sc_scatter(e, dsts3, zeros_init):
    return _build_sc_scatter(dsts3.shape[1])(e, dsts3, zeros_init)


T_NODE = 2000
T_EDGE = 2048


def _dot(a, b):
    return jnp.dot(a, b, preferred_element_type=F32)


def _node_enc_body(nf, w1, b1, w2, b2, w3, b3, wd, ws, x_o, xd_o, xs_o):
    h = jnp.maximum(_dot(nf[...], w1[...]) + b1[...], 0.0)
    h = jnp.maximum(_dot(h, w2[...]) + b2[...], 0.0)
    x = _dot(h, w3[...]) + b3[...]
    x_o[...] = x
    xd_o[...] = _dot(x, wd[...])
    xs_o[...] = _dot(x, ws[...])


def _edge_enc_body(ef, w1, b1, w2, b2, w3, b3, e_o):
    h = jnp.maximum(_dot(ef[...], w1[...]) + b1[...], 0.0)
    h = jnp.maximum(_dot(h, w2[...]) + b2[...], 0.0)
    e_o[...] = _dot(h, w3[...]) + b3[...]


def _edge_mlp_body(g, e, we, b1, w2, b2, w3, b3, e_o):
    h = g[...] + _dot(e[...], we[...]) + b1[...]
    h = jnp.maximum(h, 0.0)
    h = jnp.maximum(_dot(h, w2[...]) + b2[...], 0.0)
    e_o[...] = _dot(h, w3[...]) + b3[...]


def _node_upd_proj_body(x, ag, ag2, wnx, wna, b1, w2, b2, w3, b3, wd, ws,
                        x_o, xd_o, xs_o):
    a = ag[0] + ag[1] + ag2[0] + ag2[1]
    h = jnp.maximum(_dot(x[...], wnx[...]) + _dot(a, wna[...]) + b1[...], 0.0)
    h = jnp.maximum(_dot(h, w2[...]) + b2[...], 0.0)
    xn = _dot(h, w3[...]) + b3[...]
    x_o[...] = xn
    xd_o[...] = _dot(xn, wd[...])
    xs_o[...] = _dot(xn, ws[...])


def _node_upd_dec_body(x, ag, ag2, wnx, wna, b1, w2, b2, w3, b3,
                       dw1, db1, dw2, db2, dw3, db3, out_o):
    a = ag[0] + ag[1] + ag2[0] + ag2[1]
    h = jnp.maximum(_dot(x[...], wnx[...]) + _dot(a, wna[...]) + b1[...], 0.0)
    h = jnp.maximum(_dot(h, w2[...]) + b2[...], 0.0)
    xn = _dot(h, w3[...]) + b3[...]
    d = jnp.maximum(_dot(xn, dw1[...]) + db1[...], 0.0)
    d = jnp.maximum(_dot(d, dw2[...]) + db2[...], 0.0)
    out_o[...] = _dot(d, dw3[...]) + db3[...]


def _row_spec(t, width):
    return pl.BlockSpec((t, width), lambda i: (i, 0))


def _w_spec(shape):
    nz = (0,) * len(shape)
    return pl.BlockSpec(shape, lambda i, _nz=nz: _nz)


def _tc_call(body, grid, in_specs, out_specs, out_shape):
    return pl.pallas_call(
        body,
        grid=(grid,),
        in_specs=in_specs,
        out_specs=out_specs,
        out_shape=out_shape,
    )


def _wb(layer):
    return layer["W"], layer["b"].reshape(1, -1)


def kernel(node_features, edge_features, edge_index, params):
    src = edge_index[0]
    dst = edge_index[1]
    npad = E_PAD - N_EDGES
    pad_ids = (jnp.arange(npad, dtype=jnp.int32) * 8) % N_NODES
    eh = E_PAD // 2
    nchunk_h = eh // NW // CSZ
    dstg = jnp.concatenate([dst, pad_ids])
    srcg = jnp.concatenate([src, pad_ids])
    dsts = jnp.concatenate([dst, jnp.full((npad,), TRASH, jnp.int32)])
    dstg3 = [dstg[h * eh:(h + 1) * eh].reshape(NW, nchunk_h, CSZ)
             for h in range(2)]
    srcg3 = [srcg[h * eh:(h + 1) * eh].reshape(NW, nchunk_h, CSZ)
             for h in range(2)]
    dsts3 = [dsts[h * eh:(h + 1) * eh].reshape(NW, nchunk_h, CSZ)
             for h in range(2)]
    ef_pad = jnp.pad(edge_features, ((0, npad), (0, 0)))
    ef_h = [ef_pad[h * eh:(h + 1) * eh] for h in range(2)]
    zeros_init = jnp.zeros((N_ACC, D_LAT), F32)

    en = params["enc_node"]
    ee = params["enc_edge"]
    proc = params["proc"]
    dec = params["dec"]

    def esplit(p):
        w1 = p[0]["W"]
        return w1[0:D_LAT], w1[D_LAT:2 * D_LAT], w1[2 * D_LAT:3 * D_LAT]

    def nsplit(p):
        w1 = p[0]["W"]
        return w1[0:D_LAT], w1[D_LAT:2 * D_LAT]

    wsp = _w_spec((D_LAT, D_LAT))
    bsp = _w_spec((1, D_LAT))

    wd0, ws0, _ = esplit(proc[0]["edge"])
    w1, b1 = _wb(en[0]); w2, b2 = _wb(en[1]); w3, b3 = _wb(en[2])
    ng = N_NODES // T_NODE
    x, xd, xs = _tc_call(
        _node_enc_body, ng,
        [_row_spec(T_NODE, D_LAT)] + [wsp, bsp] * 3 + [wsp, wsp],
        [_row_spec(T_NODE, D_LAT)] * 3,
        [jax.ShapeDtypeStruct((N_NODES, D_LAT), F32)] * 3,
    )(node_features, w1, b1, w2, b2, w3, b3, wd0, ws0)

    w1, b1 = _wb(ee[0]); w2, b2 = _wb(ee[1]); w3, b3 = _wb(ee[2])
    eg = eh // T_EDGE
    e_h = [
        _tc_call(
            _edge_enc_body, eg,
            [_row_spec(T_EDGE, 16), _w_spec((16, D_LAT)), bsp, wsp, bsp,
             wsp, bsp],
            _row_spec(T_EDGE, D_LAT),
            jax.ShapeDtypeStruct((eh, D_LAT), F32),
        )(ef_h[h], w1, b1, w2, b2, w3, b3)
        for h in range(2)
    ]

    n_steps = len(proc)
    out = None
    for i in range(n_steps):
        pe = proc[i]["edge"]
        pn = proc[i]["node"]
        _, _, we = esplit(pe)
        b1e = pe[0]["b"].reshape(1, -1)
        w2e, b2e = _wb(pe[1]); w3e, b3e = _wb(pe[2])

        g_h = [_sc_gather(xd, xs, dstg3[h], srcg3[h]) for h in range(2)]
        ag_h = []
        for h in range(2):
            e_h[h] = _tc_call(
                _edge_mlp_body, eg,
                [_row_spec(T_EDGE, D_LAT)] * 2 + [wsp, bsp, wsp, bsp, wsp,
                                                  bsp],
                _row_spec(T_EDGE, D_LAT),
                jax.ShapeDtypeStruct((eh, D_LAT), F32),
            )(g_h[h], e_h[h], we, b1e, w2e, b2e, w3e, b3e)
            ag_h.append(_sc_scatter(e_h[h], dsts3[h], zeros_init))

        wnx, wna = nsplit(pn)
        b1n = pn[0]["b"].reshape(1, -1)
        w2n, b2n = _wb(pn[1]); w3n, b3n = _wb(pn[2])
        agspec = pl.BlockSpec((NC, T_NODE, D_LAT), lambda i: (0, i, 0))

        if i + 1 < n_steps:
            wd1, ws1, _ = esplit(proc[i + 1]["edge"])
            x, xd, xs = _tc_call(
                _node_upd_proj_body, ng,
                [_row_spec(T_NODE, D_LAT), agspec, agspec,
                 wsp, wsp, bsp, wsp, bsp, wsp, bsp, wsp, wsp],
                [_row_spec(T_NODE, D_LAT)] * 3,
                [jax.ShapeDtypeStruct((N_NODES, D_LAT), F32)] * 3,
            )(x, ag_h[0], ag_h[1], wnx, wna, b1n, w2n, b2n, w3n, b3n,
              wd1, ws1)
        else:
            dw1, db1 = _wb(dec[0]); dw2, db2 = _wb(dec[1]); dw3, db3 = _wb(dec[2])
            out = _tc_call(
                _node_upd_dec_body, ng,
                [_row_spec(T_NODE, D_LAT), agspec, agspec,
                 wsp, wsp, bsp, wsp, bsp, wsp, bsp,
                 wsp, bsp, wsp, bsp, _w_spec((D_LAT, 3)), _w_spec((1, 3))],
                _row_spec(T_NODE, 3),
                jax.ShapeDtypeStruct((N_NODES, 3), F32),
            )(x, ag_h[0], ag_h[1], wnx, wna, b1n, w2n, b2n, w3n, b3n,
              dw1, db1, dw2, db2, dw3, db3)
    return out


# speedup vs baseline: 1.5122x; 1.0007x over previous
import functools

import jax
import jax.numpy as jnp
from jax import lax
from jax.experimental import pallas as pl
from jax.experimental.pallas import tpu as pltpu
from jax.experimental.pallas import tpu_sc as plsc

N_NODES = 10000
N_EDGES = 320000
D_LAT = 128

NC = 2
NS = 16
NW = NC * NS
CSZ = 128
NCHUNK = 80
EPW = NCHUNK * CSZ
E_PAD = NW * EPW
N_ACC = 10016
TRASH = N_NODES
RPT = 624
REM = N_ACC - NS * RPT
REM_OFF = NS * RPT

F32 = jnp.float32
BF16 = jnp.bfloat16


def _mesh():
    return plsc.VectorSubcoreMesh(
        core_axis_name="c", subcore_axis_name="s", num_cores=NC, num_subcores=NS
    )


@functools.cache
def _build_sc_gather(nchunk):
    epw = nchunk * CSZ
    ne = NW * epw

    @functools.partial(
        pl.kernel,
        out_type=jax.ShapeDtypeStruct((ne, D_LAT), F32),
        mesh=_mesh(),
        scratch_types=[
            pltpu.VMEM((nchunk, CSZ), jnp.int32),
            pltpu.VMEM((nchunk, CSZ), jnp.int32),
            pltpu.VMEM((CSZ, D_LAT), F32),
            pltpu.VMEM((CSZ, D_LAT), F32),
            pltpu.VMEM((CSZ, D_LAT), F32),
            pltpu.VMEM((CSZ, D_LAT), F32),
            pltpu.VMEM((CSZ, D_LAT), F32),
            pltpu.VMEM((CSZ, D_LAT), F32),
            pltpu.SemaphoreType.DMA,
            pltpu.SemaphoreType.DMA,
            pltpu.SemaphoreType.DMA,
            pltpu.SemaphoreType.DMA,
        ],
    )
    def gather(xd_hbm, xs_hbm, dst_hbm, src_hbm, g_hbm,
               idxd_v, idxs_v, r1a, r1b, r2a, r2b, sa, sb,
               gsa, gsb, wsa, wsb):
        wid = lax.axis_index("s") * NC + lax.axis_index("c")
        base = wid * epw
        R1 = (r1a, r1b)
        R2 = (r2a, r2b)
        S = (sa, sb)
        GS = (gsa, gsb)
        WS = (wsa, wsb)
        pltpu.sync_copy(dst_hbm.at[wid], idxd_v)
        pltpu.sync_copy(src_hbm.at[wid], idxs_v)

        def fire_gather(k, b):
            pltpu.async_copy(xd_hbm.at[idxd_v.at[k]], R1[b], GS[b])
            pltpu.async_copy(xs_hbm.at[idxs_v.at[k]], R2[b], GS[b])

        def drain_gather(b):
            pltpu.make_async_copy(xd_hbm.at[idxd_v.at[0]], R1[b], GS[b]).wait()
            pltpu.make_async_copy(xs_hbm.at[idxs_v.at[0]], R2[b], GS[b]).wait()

        def fire_write(k, b):
            off = base + k * CSZ
            pltpu.async_copy(S[b], g_hbm.at[pl.ds(off, CSZ)], WS[b])

        def drain_write(b):
            pltpu.make_async_copy(S[b], g_hbm.at[pl.ds(base, CSZ)], WS[b]).wait()

        def add_rows(b):
            r1 = R1[b]
            r2 = R2[b]
            sm = S[b]

            def row(r, carry):
                for cidx in range(D_LAT // 16):
                    sl = pl.ds(cidx * 16, 16)
                    sm[r, sl] = r1[r, sl] + r2[r, sl]
                return carry

            lax.fori_loop(0, CSZ, row, 0)

        fire_gather(0, 0)

        @pl.loop(0, nchunk, step=2)
        def _pair(j):
            for b in range(2):
                k = j + b
                o = 1 - b
                drain_gather(b)

                @pl.when(k + 1 < nchunk)
                def _():
                    fire_gather(k + 1, o)

                @pl.when(k > 1)
                def _():
                    drain_write(b)

                add_rows(b)
                fire_write(k, b)

        drain_write(0)
        drain_write(1)

    return gather


def _sc_gather(xd, xs, dstg3, srcg3):
    return _build_sc_gather(dstg3.shape[1])(xd, xs, dstg3, srcg3)


@functools.cache
def _build_sc_scatter(nchunk):
    epw = nchunk * CSZ

    @functools.partial(
        pl.kernel,
        out_type=jax.ShapeDtypeStruct((NC, N_ACC, D_LAT), F32),
        mesh=_mesh(),
        scratch_types=[
            pltpu.VMEM((nchunk, CSZ), jnp.int32),
            pltpu.VMEM((CSZ, D_LAT), F32),
            pltpu.VMEM((CSZ, D_LAT), F32),
            pltpu.VMEM_SHARED((N_ACC, D_LAT), F32),
            pltpu.SemaphoreType.DMA,
            pltpu.SemaphoreType.DMA,
            pltpu.SemaphoreType.DMA,
            pltpu.SemaphoreType.DMA,
        ],
    )
    def scatter(e_hbm, dst_hbm, z_hbm, out_hbm,
                idx_v, ra, rb, acc_sh, rsa, rsb, asa, asb):
        c = lax.axis_index("c")
        s = lax.axis_index("s")
        wid = s * NC + c
        base = wid * epw
        R = (ra, rb)
        RS = (rsa, rsb)
        AS = (asa, asb)

        def fire_read(k, b):
            pltpu.async_copy(e_hbm.at[pl.ds(base + k * CSZ, CSZ)], R[b], RS[b])

        def drain_read(b):
            pltpu.make_async_copy(e_hbm.at[pl.ds(base, CSZ)], R[b], RS[b]).wait()

        def fire_add(k, b):
            pltpu.async_copy(R[b], acc_sh.at[idx_v.at[k]], AS[b], add=True)

        def drain_add(b):
            pltpu.make_async_copy(R[b], acc_sh.at[idx_v.at[0]], AS[b]).wait()

        fire_read(0, 0)
        pltpu.sync_copy(dst_hbm.at[wid], idx_v)
        pltpu.sync_copy(z_hbm.at[pl.ds(s * RPT, RPT)],
                        acc_sh.at[pl.ds(s * RPT, RPT)])

        @pl.when(s == NS - 1)
        def _zero_rem():
            pltpu.sync_copy(z_hbm.at[pl.ds(REM_OFF, REM)],
                            acc_sh.at[pl.ds(REM_OFF, REM)])

        plsc.subcore_barrier()

        @pl.loop(0, nchunk, step=2)
        def _pair(j):
            for b in range(2):
                k = j + b
                o = 1 - b
                drain_read(b)

                @pl.when(k > 0)
                def _():
                    drain_add(o)

                @pl.when(k + 1 < nchunk)
                def _():
                    fire_read(k + 1, o)

                fire_add(k, b)

        drain_add((nchunk - 1) % 2)
        plsc.subcore_barrier()
        pltpu.sync_copy(acc_sh.at[pl.ds(s * RPT, RPT)],
                        out_hbm.at[c, pl.ds(s * RPT, RPT)])

        @pl.when(s == NS - 1)
        def _write_rem():
            pltpu.sync_copy(acc_sh.at[pl.ds(REM_OFF, REM)],
                            out_hbm.at[c, pl.ds(REM_OFF, REM)])

    return scatter


def _sc_scatter(e, dsts3, zeros_init):
    return _build_sc_scatter(dsts3.shape[1])(e, dsts3, zeros_init)


T_NODE = 2000
T_EDGE = 2048


def _dot(a, b):
    return jnp.dot(a, b, preferred_element_type=F32)


def _node_enc_body(nf, w1, b1, w2, b2, w3, b3, wd, ws, x_o, xd_o, xs_o):
    h = jnp.maximum(_dot(nf[...], w1[...]) + b1[...], 0.0)
    h = jnp.maximum(_dot(h, w2[...]) + b2[...], 0.0)
    x = _dot(h, w3[...]) + b3[...]
    x_o[...] = x
    xd_o[...] = _dot(x, wd[...])
    xs_o[...] = _dot(x, ws[...])


def _edge_enc_body(ef, w1, b1, w2, b2, w3, b3, e_o):
    h = jnp.maximum(_dot(ef[...], w1[...]) + b1[...], 0.0)
    h = jnp.maximum(_dot(h, w2[...]) + b2[...], 0.0)
    e_o[...] = _dot(h, w3[...]) + b3[...]


def _edge_mlp_body(g, e, we, b1, w2, b2, w3, b3, e_o):
    h = g[...] + _dot(e[...], we[...]) + b1[...]
    h = jnp.maximum(h, 0.0)
    h = jnp.maximum(_dot(h, w2[...]) + b2[...], 0.0)
    e_o[...] = _dot(h, w3[...]) + b3[...]


def _node_upd_proj_body(x, ag, ag2, wnx, wna, b1, w2, b2, w3, b3, wd, ws,
                        x_o, xd_o, xs_o):
    a = ag[0] + ag[1] + ag2[0] + ag2[1]
    h = jnp.maximum(_dot(x[...], wnx[...]) + _dot(a, wna[...]) + b1[...], 0.0)
    h = jnp.maximum(_dot(h, w2[...]) + b2[...], 0.0)
    xn = _dot(h, w3[...]) + b3[...]
    x_o[...] = xn
    xd_o[...] = _dot(xn, wd[...])
    xs_o[...] = _dot(xn, ws[...])


def _node_upd_dec_body(x, ag, ag2, wnx, wna, b1, w2, b2, w3, b3,
                       dw1, db1, dw2, db2, dw3, db3, out_o):
    a = ag[0] + ag[1] + ag2[0] + ag2[1]
    h = jnp.maximum(_dot(x[...], wnx[...]) + _dot(a, wna[...]) + b1[...], 0.0)
    h = jnp.maximum(_dot(h, w2[...]) + b2[...], 0.0)
    xn = _dot(h, w3[...]) + b3[...]
    d = jnp.maximum(_dot(xn, dw1[...]) + db1[...], 0.0)
    d = jnp.maximum(_dot(d, dw2[...]) + db2[...], 0.0)
    out_o[...] = _dot(d, dw3[...]) + db3[...]


def _row_spec(t, width):
    return pl.BlockSpec((t, width), lambda i: (i, 0))


def _w_spec(shape):
    nz = (0,) * len(shape)
    return pl.BlockSpec(shape, lambda i, _nz=nz: _nz)


def _tc_call(body, grid, in_specs, out_specs, out_shape):
    return pl.pallas_call(
        body,
        grid=(grid,),
        in_specs=in_specs,
        out_specs=out_specs,
        out_shape=out_shape,
    )


def _wb(layer):
    return layer["W"], layer["b"].reshape(1, -1)


def kernel(node_features, edge_features, edge_index, params):
    src = edge_index[0]
    dst = edge_index[1]
    npad = E_PAD - N_EDGES
    pad_ids = (jnp.arange(npad, dtype=jnp.int32) * 8) % N_NODES
    eh = E_PAD // 2
    nchunk_h = eh // NW // CSZ
    dstg = jnp.concatenate([dst, pad_ids])
    srcg = jnp.concatenate([src, pad_ids])
    dsts = jnp.concatenate([dst, jnp.full((npad,), TRASH, jnp.int32)])
    dstg3 = [dstg[h * eh:(h + 1) * eh].reshape(NW, nchunk_h, CSZ)
             for h in range(2)]
    srcg3 = [srcg[h * eh:(h + 1) * eh].reshape(NW, nchunk_h, CSZ)
             for h in range(2)]
    dsts3 = [dsts[h * eh:(h + 1) * eh].reshape(NW, nchunk_h, CSZ)
             for h in range(2)]
    ef_pad = jnp.pad(edge_features, ((0, npad), (0, 0)))
    ef_h = [ef_pad[h * eh:(h + 1) * eh] for h in range(2)]
    zeros_init = jnp.zeros((N_ACC, D_LAT), F32)

    en = params["enc_node"]
    ee = params["enc_edge"]
    proc = params["proc"]
    dec = params["dec"]

    def esplit(p):
        w1 = p[0]["W"]
        return w1[0:D_LAT], w1[D_LAT:2 * D_LAT], w1[2 * D_LAT:3 * D_LAT]

    def nsplit(p):
        w1 = p[0]["W"]
        return w1[0:D_LAT], w1[D_LAT:2 * D_LAT]

    wsp = _w_spec((D_LAT, D_LAT))
    bsp = _w_spec((1, D_LAT))

    wd0, ws0, _ = esplit(proc[0]["edge"])
    w1, b1 = _wb(en[0]); w2, b2 = _wb(en[1]); w3, b3 = _wb(en[2])
    ng = N_NODES // T_NODE
    x, xd, xs = _tc_call(
        _node_enc_body, ng,
        [_row_spec(T_NODE, D_LAT)] + [wsp, bsp] * 3 + [wsp, wsp],
        [_row_spec(T_NODE, D_LAT)] * 3,
        [jax.ShapeDtypeStruct((N_NODES, D_LAT), F32)] * 3,
    )(node_features, w1, b1, w2, b2, w3, b3, wd0, ws0)

    w1, b1 = _wb(ee[0]); w2, b2 = _wb(ee[1]); w3, b3 = _wb(ee[2])
    eg = eh // T_EDGE
    e_h = [
        _tc_call(
            _edge_enc_body, eg,
            [_row_spec(T_EDGE, 16), _w_spec((16, D_LAT)), bsp, wsp, bsp,
             wsp, bsp],
            _row_spec(T_EDGE, D_LAT),
            jax.ShapeDtypeStruct((eh, D_LAT), F32),
        )(ef_h[h], w1, b1, w2, b2, w3, b3)
        for h in range(2)
    ]

    n_steps = len(proc)
    out = None
    for i in range(n_steps):
        pe = proc[i]["edge"]
        pn = proc[i]["node"]
        _, _, we = esplit(pe)
        b1e = pe[0]["b"].reshape(1, -1)
        w2e, b2e = _wb(pe[1]); w3e, b3e = _wb(pe[2])

        g_h = [_sc_gather(xd, xs, dstg3[h], srcg3[h]) for h in range(2)]
        ag_h = []
        for h in range(2):
            e_h[h] = _tc_call(
                _edge_mlp_body, eg,
                [_row_spec(T_EDGE, D_LAT)] * 2 + [wsp, bsp, wsp, bsp, wsp,
                                                  bsp],
                _row_spec(T_EDGE, D_LAT),
                jax.ShapeDtypeStruct((eh, D_LAT), F32),
            )(g_h[h], e_h[h], we, b1e, w2e, b2e, w3e, b3e)
            ag_h.append(_sc_scatter(e_h[h], dsts3[h], zeros_init))

        wnx, wna = nsplit(pn)
        b1n = pn[0]["b"].reshape(1, -1)
        w2n, b2n = _wb(pn[1]); w3n, b3n = _wb(pn[2])
        agspec = pl.BlockSpec((NC, T_NODE, D_LAT), lambda i: (0, i, 0))

        if i + 1 < n_steps:
            wd1, ws1, _ = esplit(proc[i + 1]["edge"])
            x, xd, xs = _tc_call(
                _node_upd_proj_body, ng,
                [_row_spec(T_NODE, D_LAT), agspec, agspec,
                 wsp, wsp, bsp, wsp, bsp, wsp, bsp, wsp, wsp],
                [_row_spec(T_NODE, D_LAT)] * 3,
                [jax.ShapeDtypeStruct((N_NODES, D_LAT), F32)] * 3,
            )(x, ag_h[0], ag_h[1], wnx, wna, b1n, w2n, b2n, w3n, b3n,
              wd1, ws1)
        else:
            dw1, db1 = _wb(dec[0]); dw2, db2 = _wb(dec[1]); dw3, db3 = _wb(dec[2])
            out = _tc_call(
                _node_upd_dec_body, ng,
                [_row_spec(T_NODE, D_LAT), agspec, agspec,
                 wsp, wsp, bsp, wsp, bsp, wsp, bsp,
                 wsp, bsp, wsp, bsp, _w_spec((D_LAT, 3)), _w_spec((1, 3))],
                _row_spec(T_NODE, 3),
                jax.ShapeDtypeStruct((N_NODES, 3), F32),
            )(x, ag_h[0], ag_h[1], wnx, wna, b1n, w2n, b2n, w3n, b3n,
              dw1, db1, dw2, db2, dw3, db3)
    return out
